# baseline (speedup 1.0000x reference)
import functools

import jax
import jax.numpy as jnp
from jax import lax
from jax.experimental import pallas as pl
from jax.experimental.pallas import tpu as pltpu
from jax.experimental.pallas import tpu_sc as plsc

N = 10000
E = 320000
HIDDEN = 128
N_CLASSES = 16

NC = 2
NS = 16
LANES = 128
CHUNKS = 80
EP = NC * NS * CHUNKS * LANES
NP = 10240
ROWS_PER_TILE = NP // NS
GL = 40


def _sc_mesh():
    return plsc.VectorSubcoreMesh(core_axis_name="c", subcore_axis_name="s",
                                  num_cores=NC, num_subcores=NS)


@functools.partial(
    pl.kernel,
    out_type=jax.ShapeDtypeStruct((NC, NP, HIDDEN), jnp.float32),
    mesh=_sc_mesh(),
    scratch_types=[
        pltpu.VMEM((GL, LANES), jnp.int32),
        pltpu.VMEM((GL, LANES), jnp.int32),
        pltpu.VMEM((LANES, HIDDEN), jnp.float32),
        pltpu.VMEM((LANES, HIDDEN), jnp.float32),
        pltpu.VMEM_SHARED((NP, HIDDEN), jnp.float32),
        pltpu.SemaphoreType.DMA,
        pltpu.SemaphoreType.DMA,
        pltpu.SemaphoreType.DMA,
        pltpu.SemaphoreType.DMA,
    ],
)
def _sc_scatter(h_hbm, src_hbm, dst_hbm, zero_hbm, out_hbm,
                src_v, dst_v, gbuf0, gbuf1, acc, gsem0, gsem1, ssem0, ssem1):
    c = lax.axis_index("c")
    s = lax.axis_index("s")
    r0 = s * ROWS_PER_TILE
    pltpu.sync_copy(zero_hbm.at[pl.ds(r0, ROWS_PER_TILE)],
                    acc.at[pl.ds(r0, ROWS_PER_TILE)])
    plsc.subcore_barrier()

    def gather(j, buf, sem):
        pltpu.async_copy(h_hbm.at[src_v.at[j]], buf, sem)

    def gwait(j, buf, sem):
        pltpu.make_async_copy(h_hbm.at[src_v.at[j]], buf, sem).wait()

    def scat(j, buf, sem):
        pltpu.async_copy(buf, acc.at[dst_v.at[j]], sem, add=True)

    def swait(j, buf, sem):
        pltpu.make_async_copy(buf, acc.at[dst_v.at[j]], sem).wait()

    def group(g, carry):
        g0 = pl.multiple_of(g * GL, 8)
        pltpu.sync_copy(src_hbm.at[c, s, pl.ds(g0, GL)], src_v)
        pltpu.sync_copy(dst_hbm.at[c, s, pl.ds(g0, GL)], dst_v)
        gather(0, gbuf0, gsem0)
        gwait(0, gbuf0, gsem0)
        scat(0, gbuf0, ssem0)
        gather(1, gbuf1, gsem1)

        def body(k, carry):
            p = 2 * k + 1
            q = 2 * k + 2
            gwait(p, gbuf1, gsem1)
            scat(p, gbuf1, ssem1)
            swait(p - 1, gbuf0, ssem0)
            gather(q, gbuf0, gsem0)
            gwait(q, gbuf0, gsem0)
            scat(q, gbuf0, ssem0)
            swait(p, gbuf1, ssem1)
            gather(q + 1, gbuf1, gsem1)
            return carry

        lax.fori_loop(0, (GL - 2) // 2, body, 0)
        gwait(GL - 1, gbuf1, gsem1)
        scat(GL - 1, gbuf1, ssem1)
        swait(GL - 2, gbuf0, ssem0)
        swait(GL - 1, gbuf1, ssem1)
        return carry

    lax.fori_loop(0, CHUNKS // GL, group, 0)
    plsc.subcore_barrier()
    pltpu.sync_copy(acc.at[pl.ds(r0, ROWS_PER_TILE)],
                    out_hbm.at[c, pl.ds(r0, ROWS_PER_TILE)])


@functools.partial(
    pl.kernel,
    out_type=jax.ShapeDtypeStruct((NC * NS, 2, NP), jnp.float32),
    mesh=_sc_mesh(),
    compiler_params=pltpu.CompilerParams(needs_layout_passes=False),
    scratch_types=[
        pltpu.VMEM((CHUNKS, LANES), jnp.int32),
        pltpu.VMEM((CHUNKS, LANES), jnp.int32),
        pltpu.VMEM((NP,), jnp.float32),
        pltpu.VMEM((NP,), jnp.float32),
    ],
)
def _sc_hist(src_hbm, dst_hbm, out_hbm, src_v, dst_v, hist_s, hist_d):
    c = lax.axis_index("c")
    s = lax.axis_index("s")
    t = s * NC + c
    pltpu.sync_copy(src_hbm.at[c, s], src_v)
    pltpu.sync_copy(dst_hbm.at[c, s], dst_v)

    zeros = jnp.zeros((16,), jnp.float32)

    def zbody(i, carry):
        hist_s[pl.ds(i * 16, 16)] = zeros
        hist_d[pl.ds(i * 16, 16)] = zeros
        return carry

    lax.fori_loop(0, NP // 16, zbody, 0)

    ones = jnp.ones((16,), jnp.float32)

    def body(i, carry):
        j = i // 8
        off = (i % 8) * 16
        plsc.addupdate_scatter(hist_s, [src_v[j, pl.ds(off, 16)]], ones)
        plsc.addupdate_scatter(hist_d, [dst_v[j, pl.ds(off, 16)]], ones)
        return carry

    lax.fori_loop(0, CHUNKS * 8, body, 0)
    pltpu.sync_copy(hist_s, out_hbm.at[t, 0])
    pltpu.sync_copy(hist_d, out_hbm.at[t, 1])


def _rsqrt16(d):
    x = plsc.bitcast(jnp.int32(0x5F3759DF) - (plsc.bitcast(d, jnp.int32) >> 1),
                     jnp.float32)
    for _ in range(3):
        x = x * (1.5 - 0.5 * d * x * x)
    return x


EBLK = 128


@functools.partial(
    pl.kernel,
    out_type=jax.ShapeDtypeStruct((2, NP, HIDDEN), jnp.float32),
    mesh=_sc_mesh(),
    compiler_params=pltpu.CompilerParams(needs_layout_passes=False),
    scratch_types=[
        pltpu.VMEM((NC * NS, ROWS_PER_TILE), jnp.float32),
        pltpu.VMEM((ROWS_PER_TILE,), jnp.float32),
        pltpu.VMEM((EBLK, HIDDEN), jnp.float32),
        pltpu.SemaphoreType.DMA,
    ],
)
def _sc_norms(parts_hbm, out_hbm, red_v, nrm_v, ebuf, sem):
    c = lax.axis_index("c")
    s = lax.axis_index("s")
    r0 = s * ROWS_PER_TILE

    for k in range(NC * NS):
        pltpu.async_copy(parts_hbm.at[k, c, pl.ds(r0, ROWS_PER_TILE)],
                         red_v.at[k], sem)
    for k in range(NC * NS):
        pltpu.make_async_copy(parts_hbm.at[k, c, pl.ds(r0, ROWS_PER_TILE)],
                              red_v.at[k], sem).wait()

    def rbody(ci, carry):
        acc = jnp.zeros((16,), jnp.float32)
        for k in range(NC * NS):
            acc = acc + red_v[k, pl.ds(ci * 16, 16)]
        nrm = jnp.where(acc > 0, _rsqrt16(acc), 1.0)
        nrm_v[pl.ds(ci * 16, 16)] = nrm
        return carry

    lax.fori_loop(0, ROWS_PER_TILE // 16, rbody, 0)

    def ebody(bi, carry):
        for g16 in range(EBLK // 16):
            nv = nrm_v[pl.ds(bi * EBLK + g16 * 16, 16)]
            for e in range(16):
                row16 = jnp.full((16,), nv[e], jnp.float32)
                for q in range(HIDDEN // 16):
                    ebuf[g16 * 16 + e, pl.ds(q * 16, 16)] = row16
        pltpu.sync_copy(ebuf, out_hbm.at[c, pl.ds(r0 + bi * EBLK, EBLK)])
        return carry

    lax.fori_loop(0, ROWS_PER_TILE // EBLK, ebody, 0)


def _norms(t):
    return t[0], t[1]


def _dot(x, w):
    return lax.dot_general(x, w, (((1,), (0,)), ((), ())),
                           preferred_element_type=jnp.float32)


def _tc_first(feat_ref, degs_ref, o_ref):
    ns, _ = _norms(degs_ref[...])
    o_ref[...] = feat_ref[...] * ns


def _tc_mid(s_ref, degs_ref, b_ref, w_ref, o_ref):
    ns, nd = _norms(degs_ref[...])
    agg = _dot(s_ref[0] + s_ref[1], w_ref[...])
    x = jnp.maximum(agg * nd + b_ref[...], 0.0)
    o_ref[...] = x * ns


def _tc_final(s_ref, degs_ref, b_ref, w_ref, o_ref):
    _, nd = _norms(degs_ref[...])
    agg = _dot(s_ref[0] + s_ref[1], w_ref[...])
    o_ref[...] = agg * nd[:, :N_CLASSES] + b_ref[...]


NPB = 5120


def _tc_call(body, out_width, *args):
    grid = NP // NPB
    in_specs = []
    for a in args:
        if a.ndim == 3 and a.shape[1] == NP:
            in_specs.append(pl.BlockSpec((NC, NPB, a.shape[2]),
                                         lambda i: (0, i, 0)))
        elif a.ndim == 2 and a.shape[0] == NP:
            in_specs.append(pl.BlockSpec((NPB, a.shape[1]),
                                         lambda i: (i, 0)))
        else:
            in_specs.append(pl.BlockSpec(a.shape, lambda i: (0,) * a.ndim))
    return pl.pallas_call(
        body,
        grid=(grid,),
        in_specs=in_specs,
        out_specs=pl.BlockSpec((NPB, out_width), lambda i: (i, 0)),
        out_shape=jax.ShapeDtypeStruct((NP, out_width), jnp.float32),
    )(*args)


def kernel(features, edge_index, W1, b1, W2, b2, W3, b3, W4, b4, W5, b5):
    pad_idx = N + jnp.arange(EP - E, dtype=jnp.int32) % (NP - N)
    src = jnp.concatenate([edge_index[0], pad_idx])
    dst = jnp.concatenate([edge_index[1], pad_idx])
    src_r = src.reshape(NC, NS, CHUNKS, LANES)
    dst_r = dst.reshape(NC, NS, CHUNKS, LANES)
    feat_pad = jnp.pad(features, ((0, NP - N), (0, 0)))

    zero128 = jnp.zeros((NP, HIDDEN), jnp.float32)

    parts = _sc_hist(src_r, dst_r)
    degs = _sc_norms(parts)

    a = _tc_call(_tc_first, HIDDEN, feat_pad, degs)
    for b_prev, w in ((b1, W1), (b2, W2), (b3, W3), (b4, W4)):
        s_part = _sc_scatter(a, src_r, dst_r, zero128)
        a = _tc_call(_tc_mid, HIDDEN, s_part, degs,
                     b_prev.reshape(1, HIDDEN), w)
    s_part = _sc_scatter(a, src_r, dst_r, zero128)
    out = _tc_call(_tc_final, N_CLASSES, s_part, degs,
                   b5.reshape(1, N_CLASSES), W5)
    return out[:N]


# speedup vs baseline: 1.0073x; 1.0073x over previous
import functools

import jax
import jax.numpy as jnp
from jax import lax
from jax.experimental import pallas as pl
from jax.experimental.pallas import tpu as pltpu
from jax.experimental.pallas import tpu_sc as plsc

N = 10000
E = 320000
HIDDEN = 128
N_CLASSES = 16

NC = 2
NS = 16
LANES = 128
CHUNKS = 80
EP = NC * NS * CHUNKS * LANES
NP = 10240
ROWS_PER_TILE = NP // NS
GL = 40


def _sc_mesh():
    return plsc.VectorSubcoreMesh(core_axis_name="c", subcore_axis_name="s",
                                  num_cores=NC, num_subcores=NS)


@functools.partial(
    pl.kernel,
    out_type=jax.ShapeDtypeStruct((NC, NP, HIDDEN), jnp.float32),
    mesh=_sc_mesh(),
    scratch_types=[
        pltpu.VMEM((GL, LANES), jnp.int32),
        pltpu.VMEM((GL, LANES), jnp.int32),
        pltpu.VMEM((LANES, HIDDEN), jnp.float32),
        pltpu.VMEM((LANES, HIDDEN), jnp.float32),
        pltpu.VMEM_SHARED((NP, HIDDEN), jnp.float32),
        pltpu.SemaphoreType.DMA,
        pltpu.SemaphoreType.DMA,
        pltpu.SemaphoreType.DMA,
        pltpu.SemaphoreType.DMA,
    ],
)
def _sc_scatter(h_hbm, src_hbm, dst_hbm, zero_hbm, out_hbm,
                src_v, dst_v, gbuf0, gbuf1, acc, gsem0, gsem1, ssem0, ssem1):
    c = lax.axis_index("c")
    s = lax.axis_index("s")
    r0 = s * ROWS_PER_TILE
    pltpu.sync_copy(zero_hbm.at[pl.ds(r0, ROWS_PER_TILE)],
                    acc.at[pl.ds(r0, ROWS_PER_TILE)])
    plsc.subcore_barrier()

    def gather(j, buf, sem):
        pltpu.async_copy(h_hbm.at[src_v.at[j]], buf, sem)

    def gwait(j, buf, sem):
        pltpu.make_async_copy(h_hbm.at[src_v.at[j]], buf, sem).wait()

    def scat(j, buf, sem):
        pltpu.async_copy(buf, acc.at[dst_v.at[j]], sem, add=True)

    def swait(j, buf, sem):
        pltpu.make_async_copy(buf, acc.at[dst_v.at[j]], sem).wait()

    def group(g, carry):
        g0 = pl.multiple_of(g * GL, 8)
        pltpu.sync_copy(src_hbm.at[c, s, pl.ds(g0, GL)], src_v)
        pltpu.sync_copy(dst_hbm.at[c, s, pl.ds(g0, GL)], dst_v)
        gather(0, gbuf0, gsem0)
        gwait(0, gbuf0, gsem0)
        scat(0, gbuf0, ssem0)
        gather(1, gbuf1, gsem1)

        def body(k, carry):
            p = 2 * k + 1
            q = 2 * k + 2
            gwait(p, gbuf1, gsem1)
            scat(p, gbuf1, ssem1)
            swait(p - 1, gbuf0, ssem0)
            gather(q, gbuf0, gsem0)
            gwait(q, gbuf0, gsem0)
            scat(q, gbuf0, ssem0)
            swait(p, gbuf1, ssem1)
            gather(q + 1, gbuf1, gsem1)
            return carry

        lax.fori_loop(0, (GL - 2) // 2, body, 0)
        gwait(GL - 1, gbuf1, gsem1)
        scat(GL - 1, gbuf1, ssem1)
        swait(GL - 2, gbuf0, ssem0)
        swait(GL - 1, gbuf1, ssem1)
        return carry

    lax.fori_loop(0, CHUNKS // GL, group, 0)
    plsc.subcore_barrier()
    pltpu.sync_copy(acc.at[pl.ds(r0, ROWS_PER_TILE)],
                    out_hbm.at[c, pl.ds(r0, ROWS_PER_TILE)])


def _rsqrt16(d):
    x = plsc.bitcast(jnp.int32(0x5F3759DF) - (plsc.bitcast(d, jnp.int32) >> 1),
                     jnp.float32)
    for _ in range(3):
        x = x * (1.5 - 0.5 * d * x * x)
    return x


EBLK = 128


@functools.partial(
    pl.kernel,
    out_type=jax.ShapeDtypeStruct((2, NP, HIDDEN), jnp.float32),
    mesh=_sc_mesh(),
    compiler_params=pltpu.CompilerParams(needs_layout_passes=False),
    scratch_types=[
        pltpu.VMEM((CHUNKS, LANES), jnp.int32),
        pltpu.VMEM((NP,), jnp.float32),
        pltpu.VMEM((NS, ROWS_PER_TILE), jnp.float32),
        pltpu.VMEM((ROWS_PER_TILE,), jnp.float32),
        pltpu.VMEM((EBLK, HIDDEN), jnp.float32),
        pltpu.VMEM_SHARED((NS, NP), jnp.float32),
        pltpu.SemaphoreType.DMA,
    ],
)
def _sc_degnorm(idx_hbm, out_hbm, idx_v, hist, red_v, nrm_v, ebuf, stage, sem):
    c = lax.axis_index("c")
    s = lax.axis_index("s")
    r0 = s * ROWS_PER_TILE

    zeros = jnp.zeros((16,), jnp.float32)

    def zbody(i, carry):
        hist[pl.ds(i * 16, 16)] = zeros
        return carry

    lax.fori_loop(0, NP // 16, zbody, 0)

    ones = jnp.ones((16,), jnp.float32)
    for g in range(2):
        pltpu.sync_copy(idx_hbm.at[c, s * 2 + g], idx_v)

        def body(i, carry):
            j = i // 8
            off = (i % 8) * 16
            plsc.addupdate_scatter(hist, [idx_v[j, pl.ds(off, 16)]], ones)
            return carry

        lax.fori_loop(0, CHUNKS * 8, body, 0)

    pltpu.sync_copy(hist, stage.at[s])
    plsc.subcore_barrier()

    for k in range(NS):
        pltpu.async_copy(stage.at[k, pl.ds(r0, ROWS_PER_TILE)],
                         red_v.at[k], sem)
    for k in range(NS):
        pltpu.make_async_copy(stage.at[k, pl.ds(r0, ROWS_PER_TILE)],
                              red_v.at[k], sem).wait()

    def rbody(ci, carry):
        acc = jnp.zeros((16,), jnp.float32)
        for k in range(NS):
            acc = acc + red_v[k, pl.ds(ci * 16, 16)]
        nrm = jnp.where(acc > 0, _rsqrt16(acc), 1.0)
        nrm_v[pl.ds(ci * 16, 16)] = nrm
        return carry

    lax.fori_loop(0, ROWS_PER_TILE // 16, rbody, 0)

    def ebody(bi, carry):
        for g16 in range(EBLK // 16):
            nv = nrm_v[pl.ds(bi * EBLK + g16 * 16, 16)]
            for e in range(16):
                row16 = jnp.full((16,), nv[e], jnp.float32)
                for q in range(HIDDEN // 16):
                    ebuf[g16 * 16 + e, pl.ds(q * 16, 16)] = row16
        pltpu.sync_copy(ebuf, out_hbm.at[c, pl.ds(r0 + bi * EBLK, EBLK)])
        return carry

    lax.fori_loop(0, ROWS_PER_TILE // EBLK, ebody, 0)


def _norms(t):
    return t[0], t[1]


def _dot(x, w):
    return lax.dot_general(x, w, (((1,), (0,)), ((), ())),
                           preferred_element_type=jnp.float32)


def _tc_first(feat_ref, degs_ref, o_ref):
    ns, _ = _norms(degs_ref[...])
    o_ref[...] = feat_ref[...] * ns


def _tc_mid(s_ref, degs_ref, b_ref, w_ref, o_ref):
    ns, nd = _norms(degs_ref[...])
    agg = _dot(s_ref[0] + s_ref[1], w_ref[...])
    x = jnp.maximum(agg * nd + b_ref[...], 0.0)
    o_ref[...] = x * ns


def _tc_final(s_ref, degs_ref, b_ref, w_ref, o_ref):
    _, nd = _norms(degs_ref[...])
    agg = _dot(s_ref[0] + s_ref[1], w_ref[...])
    o_ref[...] = agg * nd[:, :N_CLASSES] + b_ref[...]


NPB = 5120


def _tc_call(body, out_width, *args):
    grid = NP // NPB
    in_specs = []
    for a in args:
        if a.ndim == 3 and a.shape[1] == NP:
            in_specs.append(pl.BlockSpec((NC, NPB, a.shape[2]),
                                         lambda i: (0, i, 0)))
        elif a.ndim == 2 and a.shape[0] == NP:
            in_specs.append(pl.BlockSpec((NPB, a.shape[1]),
                                         lambda i: (i, 0)))
        else:
            in_specs.append(pl.BlockSpec(a.shape, lambda i: (0,) * a.ndim))
    return pl.pallas_call(
        body,
        grid=(grid,),
        in_specs=in_specs,
        out_specs=pl.BlockSpec((NPB, out_width), lambda i: (i, 0)),
        out_shape=jax.ShapeDtypeStruct((NP, out_width), jnp.float32),
    )(*args)


def kernel(features, edge_index, W1, b1, W2, b2, W3, b3, W4, b4, W5, b5):
    pad_idx = N + jnp.arange(EP - E, dtype=jnp.int32) % (NP - N)
    src = jnp.concatenate([edge_index[0], pad_idx])
    dst = jnp.concatenate([edge_index[1], pad_idx])
    src_r = src.reshape(NC, NS, CHUNKS, LANES)
    dst_r = dst.reshape(NC, NS, CHUNKS, LANES)
    feat_pad = jnp.pad(features, ((0, NP - N), (0, 0)))

    zero128 = jnp.zeros((NP, HIDDEN), jnp.float32)

    idx_both = jnp.stack([src.reshape(NC * NS, CHUNKS, LANES),
                          dst.reshape(NC * NS, CHUNKS, LANES)])
    degs = _sc_degnorm(idx_both)

    a = _tc_call(_tc_first, HIDDEN, feat_pad, degs)
    for b_prev, w in ((b1, W1), (b2, W2), (b3, W3), (b4, W4)):
        s_part = _sc_scatter(a, src_r, dst_r, zero128)
        a = _tc_call(_tc_mid, HIDDEN, s_part, degs,
                     b_prev.reshape(1, HIDDEN), w)
    s_part = _sc_scatter(a, src_r, dst_r, zero128)
    out = _tc_call(_tc_final, N_CLASSES, s_part, degs,
                   b5.reshape(1, N_CLASSES), W5)
    return out[:N]
